# Initial kernel scaffold; baseline (speedup 1.0000x reference)
import jax
import jax.numpy as jnp
from jax.experimental import pallas as pl


def kernel(features, edge_index, W, b):
    raise NotImplementedError("write your pallas kernel here")


# speedup vs baseline: 3.4596x; 3.4596x over previous
import functools

import jax
import jax.numpy as jnp
from jax import lax
from jax.experimental import pallas as pl
from jax.experimental.pallas import tpu as pltpu
from jax.experimental.pallas import tpu_sc as plsc

N_NODES = 10000
D = 128
N_PAD = 10240
NC = 2
NS = 16
NW = NC * NS
CH = 64
NCH = 160
EPW = CH * NCH
E_PAD = NW * EPW

_mesh = plsc.VectorSubcoreMesh(core_axis_name="c", subcore_axis_name="s")
_ROWS_PER_TILE = N_PAD // NS


@functools.partial(
    pl.kernel,
    out_type=jax.ShapeDtypeStruct((NC, N_PAD), jnp.float32),
    mesh=_mesh,
    scratch_types=[
        pltpu.VMEM((NCH, CH), jnp.int32),
        pltpu.VMEM((CH,), jnp.float32),
        pltpu.VMEM_SHARED((N_PAD,), jnp.float32),
    ],
)
def _sc_degs(dst_hbm, zeros1_hbm, out_hbm, dst_v, ones_v, degs_sh):
    c = lax.axis_index("c")
    s = lax.axis_index("s")
    wid = c * NS + s
    r0 = s * _ROWS_PER_TILE
    pltpu.sync_copy(zeros1_hbm.at[pl.ds(r0, _ROWS_PER_TILE)],
                    degs_sh.at[pl.ds(r0, _ROWS_PER_TILE)])
    for i in range(CH // 16):
        ones_v[pl.ds(i * 16, 16)] = jnp.ones((16,), jnp.float32)
    pltpu.sync_copy(dst_hbm.at[wid], dst_v)
    plsc.subcore_barrier()

    def body(i, carry):
        pltpu.sync_copy(ones_v, degs_sh.at[dst_v.at[i]], add=True)
        return carry

    lax.fori_loop(0, NCH, body, 0)
    plsc.subcore_barrier()
    pltpu.sync_copy(degs_sh.at[pl.ds(r0, _ROWS_PER_TILE)],
                    out_hbm.at[c, pl.ds(r0, _ROWS_PER_TILE)])


@functools.partial(
    pl.kernel,
    out_type=jax.ShapeDtypeStruct((NC, N_PAD, D), jnp.float32),
    mesh=_mesh,
    scratch_types=[
        pltpu.VMEM((4, CH), jnp.int32),
        pltpu.VMEM((4, CH), jnp.int32),
        pltpu.VMEM((CH, D), jnp.float32),
        pltpu.VMEM((CH, D), jnp.float32),
        pltpu.VMEM_SHARED((N_PAD, D), jnp.float32),
        pltpu.SemaphoreType.DMA,
        pltpu.SemaphoreType.DMA,
        pltpu.SemaphoreType.DMA,
        pltpu.SemaphoreType.DMA,
    ],
)
def _sc_agg(h_hbm, src_hbm, dst_hbm, zeros_hbm, out_hbm,
            src_b, dst_b, buf_a, buf_b, acc_sh, sem_a, sem_b, sem_i01, sem_i23):
    c = lax.axis_index("c")
    s = lax.axis_index("s")
    wid = c * NS + s
    r0 = s * _ROWS_PER_TILE
    pltpu.sync_copy(zeros_hbm.at[pl.ds(r0, _ROWS_PER_TILE)],
                    acc_sh.at[pl.ds(r0, _ROWS_PER_TILE)])
    plsc.subcore_barrier()

    def load_idx(chunk, slot, sem):
        pltpu.async_copy(src_hbm.at[wid, chunk], src_b.at[slot], sem)
        pltpu.async_copy(dst_hbm.at[wid, chunk], dst_b.at[slot], sem)

    def wait_idx_pair(sem):
        for _ in range(4):
            pltpu.make_async_copy(src_hbm.at[wid, 0], src_b.at[0], sem).wait()

    def gather(slot, buf, sem):
        pltpu.async_copy(h_hbm.at[src_b.at[slot]], buf, sem)

    def wait_gather(buf, sem):
        pltpu.make_async_copy(h_hbm.at[src_b.at[0]], buf, sem).wait()

    def scatter(slot, buf):
        pltpu.sync_copy(buf, acc_sh.at[dst_b.at[slot]], add=True)

    load_idx(0, 0, sem_i01)
    load_idx(1, 1, sem_i01)
    load_idx(2, 2, sem_i23)
    load_idx(3, 3, sem_i23)

    NJ = NCH // 4

    def body(j, carry):
        c0 = 4 * j
        wait_idx_pair(sem_i01)
        gather(0, buf_a, sem_a)
        gather(1, buf_b, sem_b)
        wait_gather(buf_a, sem_a)
        scatter(0, buf_a)
        wait_idx_pair(sem_i23)
        gather(2, buf_a, sem_a)
        wait_gather(buf_b, sem_b)
        scatter(1, buf_b)

        @pl.when(j + 1 < NJ)
        def _():
            load_idx(c0 + 4, 0, sem_i01)
            load_idx(c0 + 5, 1, sem_i01)

        gather(3, buf_b, sem_b)
        wait_gather(buf_a, sem_a)
        scatter(2, buf_a)
        wait_gather(buf_b, sem_b)
        scatter(3, buf_b)

        @pl.when(j + 1 < NJ)
        def _():
            load_idx(c0 + 6, 2, sem_i23)
            load_idx(c0 + 7, 3, sem_i23)

        return carry

    lax.fori_loop(0, NJ, body, 0)
    plsc.subcore_barrier()
    pltpu.sync_copy(acc_sh.at[pl.ds(r0, _ROWS_PER_TILE)],
                    out_hbm.at[c, pl.ds(r0, _ROWS_PER_TILE)])


def _tc_prescale_body(degs_ref, feat_ref, h_ref):
    d = degs_ref[0] + degs_ref[1]
    norm = jnp.where(d > 0.0, lax.rsqrt(d), 0.0)
    h_ref[...] = feat_ref[...] * norm[:N_NODES]


def _tc_finish_body(agg_ref, degs_ref, w_ref, b_ref, out_ref):
    a = agg_ref[0, :N_NODES] + agg_ref[1, :N_NODES]
    d = degs_ref[0, :N_NODES] + degs_ref[1, :N_NODES]
    norm = jnp.where(d > 0.0, lax.rsqrt(d), 0.0)
    h2 = a * norm
    out = lax.dot_general(h2, w_ref[...],
                          dimension_numbers=(((1,), (1,)), ((), ())),
                          preferred_element_type=jnp.float32)
    out_ref[...] = out + b_ref[...]


def kernel(features, edge_index, W, b):
    features = features.astype(jnp.float32)
    src = edge_index[0].astype(jnp.int32)
    dst = edge_index[1].astype(jnp.int32)
    n_extra = E_PAD - src.shape[0]
    src_p = jnp.concatenate([src, jnp.zeros((n_extra,), jnp.int32)])
    dst_p = jnp.concatenate([dst, jnp.full((n_extra,), N_NODES, jnp.int32)])
    src3 = src_p.reshape(NW, NCH, CH)
    dst3 = dst_p.reshape(NW, NCH, CH)
    zeros1 = jnp.zeros((N_PAD,), jnp.float32)
    zeros2 = jnp.zeros((N_PAD, D), jnp.float32)

    degs_p = _sc_degs(dst3, zeros1)
    degs_p3 = degs_p[:, :, None]

    h = pl.pallas_call(
        _tc_prescale_body,
        out_shape=jax.ShapeDtypeStruct((N_NODES, D), jnp.float32),
    )(degs_p3, features)

    agg_p = _sc_agg(h, src3, dst3, zeros2)

    out = pl.pallas_call(
        _tc_finish_body,
        out_shape=jax.ShapeDtypeStruct((N_NODES, D), jnp.float32),
    )(agg_p, degs_p3, W.astype(jnp.float32), b.reshape(1, D).astype(jnp.float32))
    return out


# speedup vs baseline: 6.8968x; 1.9935x over previous
import functools

import jax
import jax.numpy as jnp
from jax import lax
from jax.experimental import pallas as pl
from jax.experimental.pallas import tpu as pltpu
from jax.experimental.pallas import tpu_sc as plsc

N_NODES = 10000
D = 128
N_PAD = 10240
NC = 2
NS = 16
NW = NC * NS
CH = 64
NCH = 160
EPW = CH * NCH
E_PAD = NW * EPW
DH = D // 2
CHA = 128
NCHA = 160

_mesh = plsc.VectorSubcoreMesh(core_axis_name="c", subcore_axis_name="s")
_ROWS_PER_TILE = N_PAD // NS


@functools.partial(
    pl.kernel,
    out_type=jax.ShapeDtypeStruct((NC, N_PAD), jnp.float32),
    mesh=_mesh,
    scratch_types=[
        pltpu.VMEM((NCH, CH), jnp.int32),
        pltpu.VMEM((CH,), jnp.float32),
        pltpu.VMEM_SHARED((N_PAD,), jnp.float32),
    ],
)
def _sc_degs(dst_hbm, zeros1_hbm, out_hbm, dst_v, ones_v, degs_sh):
    c = lax.axis_index("c")
    s = lax.axis_index("s")
    wid = c * NS + s
    r0 = s * _ROWS_PER_TILE
    pltpu.sync_copy(zeros1_hbm.at[pl.ds(r0, _ROWS_PER_TILE)],
                    degs_sh.at[pl.ds(r0, _ROWS_PER_TILE)])
    for i in range(CH // 16):
        ones_v[pl.ds(i * 16, 16)] = jnp.ones((16,), jnp.float32)
    pltpu.sync_copy(dst_hbm.at[wid], dst_v)
    plsc.subcore_barrier()

    def body(i, carry):
        pltpu.sync_copy(ones_v, degs_sh.at[dst_v.at[i]], add=True)
        return carry

    lax.fori_loop(0, NCH, body, 0)
    plsc.subcore_barrier()
    pltpu.sync_copy(degs_sh.at[pl.ds(r0, _ROWS_PER_TILE)],
                    out_hbm.at[c, pl.ds(r0, _ROWS_PER_TILE)])


@functools.partial(
    pl.kernel,
    out_type=jax.ShapeDtypeStruct((NC, N_PAD, DH), jnp.float32),
    mesh=_mesh,
    scratch_types=[
        pltpu.VMEM((4, CHA), jnp.int32),
        pltpu.VMEM((4, CHA), jnp.int32),
        pltpu.VMEM((CHA, DH), jnp.float32),
        pltpu.VMEM((CHA, DH), jnp.float32),
        pltpu.VMEM_SHARED((N_PAD, DH), jnp.float32),
        pltpu.VMEM_SHARED((N_PAD, DH), jnp.float32),
        pltpu.SemaphoreType.DMA,
        pltpu.SemaphoreType.DMA,
        pltpu.SemaphoreType.DMA,
        pltpu.SemaphoreType.DMA,
    ],
    compiler_params=pltpu.CompilerParams(use_tc_tiling_on_sc=False),
)
def _sc_agg(h_hbm, src_hbm, dst_hbm, zeros_hbm, out_hbm,
            src_b, dst_b, buf_a, buf_b, h_sh, acc_sh,
            sem_a, sem_b, sem_i01, sem_i23):
    c = lax.axis_index("c")
    s = lax.axis_index("s")
    r0 = s * _ROWS_PER_TILE
    pltpu.sync_copy(zeros_hbm.at[pl.ds(r0, _ROWS_PER_TILE)],
                    acc_sh.at[pl.ds(r0, _ROWS_PER_TILE)])
    pltpu.sync_copy(h_hbm.at[c, pl.ds(r0, _ROWS_PER_TILE)],
                    h_sh.at[pl.ds(r0, _ROWS_PER_TILE)])
    plsc.subcore_barrier()

    def load_idx(chunk, slot, sem):
        pltpu.async_copy(src_hbm.at[s, chunk], src_b.at[slot], sem)
        pltpu.async_copy(dst_hbm.at[s, chunk], dst_b.at[slot], sem)

    def wait_idx_pair(sem):
        for _ in range(4):
            pltpu.make_async_copy(src_hbm.at[s, 0], src_b.at[0], sem).wait()

    def gather(slot, buf, sem):
        pltpu.async_copy(h_sh.at[src_b.at[slot]], buf, sem)

    def wait_gather(buf, sem):
        pltpu.make_async_copy(h_sh.at[src_b.at[0]], buf, sem).wait()

    def scatter(slot, buf):
        pltpu.sync_copy(buf, acc_sh.at[dst_b.at[slot]], add=True)

    load_idx(0, 0, sem_i01)
    load_idx(1, 1, sem_i01)
    load_idx(2, 2, sem_i23)
    load_idx(3, 3, sem_i23)

    NJ = NCHA // 4

    def body(j, carry):
        c0 = 4 * j
        wait_idx_pair(sem_i01)
        gather(0, buf_a, sem_a)
        gather(1, buf_b, sem_b)
        wait_gather(buf_a, sem_a)
        scatter(0, buf_a)
        wait_idx_pair(sem_i23)
        gather(2, buf_a, sem_a)
        wait_gather(buf_b, sem_b)
        scatter(1, buf_b)

        @pl.when(j + 1 < NJ)
        def _():
            load_idx(c0 + 4, 0, sem_i01)
            load_idx(c0 + 5, 1, sem_i01)

        gather(3, buf_b, sem_b)
        wait_gather(buf_a, sem_a)
        scatter(2, buf_a)
        wait_gather(buf_b, sem_b)
        scatter(3, buf_b)

        @pl.when(j + 1 < NJ)
        def _():
            load_idx(c0 + 6, 2, sem_i23)
            load_idx(c0 + 7, 3, sem_i23)

        return carry

    lax.fori_loop(0, NJ, body, 0)
    plsc.subcore_barrier()
    pltpu.sync_copy(acc_sh.at[pl.ds(r0, _ROWS_PER_TILE)],
                    out_hbm.at[c, pl.ds(r0, _ROWS_PER_TILE)])


def _tc_prescale_body(degs_ref, feat_ref, h_ref):
    d = degs_ref[0] + degs_ref[1]
    norm = jnp.where(d > 0.0, lax.rsqrt(d), 0.0)
    h = feat_ref[...] * norm[:N_NODES]
    h_ref[0, :N_NODES] = h[:, :DH]
    h_ref[1, :N_NODES] = h[:, DH:]
    pad = jnp.zeros((N_PAD - N_NODES, DH), jnp.float32)
    h_ref[0, N_NODES:] = pad
    h_ref[1, N_NODES:] = pad


def _tc_finish_body(agg_ref, degs_ref, w_ref, b_ref, out_ref):
    a = jnp.concatenate(
        [agg_ref[0, :N_NODES], agg_ref[1, :N_NODES]], axis=1)
    d = degs_ref[0, :N_NODES] + degs_ref[1, :N_NODES]
    norm = jnp.where(d > 0.0, lax.rsqrt(d), 0.0)
    h2 = a * norm
    out = lax.dot_general(h2, w_ref[...],
                          dimension_numbers=(((1,), (1,)), ((), ())),
                          preferred_element_type=jnp.float32)
    out_ref[...] = out + b_ref[...]


def kernel(features, edge_index, W, b):
    features = features.astype(jnp.float32)
    src = edge_index[0].astype(jnp.int32)
    dst = edge_index[1].astype(jnp.int32)
    n_extra = E_PAD - src.shape[0]
    src_p = jnp.concatenate([src, jnp.zeros((n_extra,), jnp.int32)])
    dst_p = jnp.concatenate([dst, jnp.full((n_extra,), N_NODES, jnp.int32)])
    src3 = src_p.reshape(NW, NCH, CH)
    dst3 = dst_p.reshape(NW, NCH, CH)
    src_a = src_p.reshape(NS, NCHA, CHA)
    dst_a = dst_p.reshape(NS, NCHA, CHA)
    zeros1 = jnp.zeros((N_PAD,), jnp.float32)
    zeros2 = jnp.zeros((N_PAD, DH), jnp.float32)

    degs_p = _sc_degs(dst3, zeros1)
    degs_p3 = degs_p[:, :, None]

    h = pl.pallas_call(
        _tc_prescale_body,
        out_shape=jax.ShapeDtypeStruct((NC, N_PAD, DH), jnp.float32),
    )(degs_p3, features)

    agg_p = _sc_agg(h, src_a, dst_a, zeros2)

    out = pl.pallas_call(
        _tc_finish_body,
        out_shape=jax.ShapeDtypeStruct((N_NODES, D), jnp.float32),
    )(agg_p, degs_p3, W.astype(jnp.float32), b.reshape(1, D).astype(jnp.float32))
    return out


# speedup vs baseline: 10.0958x; 1.4638x over previous
import functools

import jax
import jax.numpy as jnp
from jax import lax
from jax.experimental import pallas as pl
from jax.experimental.pallas import tpu as pltpu
from jax.experimental.pallas import tpu_sc as plsc

N_NODES = 10000
N_EDGES = 320000
D = 128
N_PAD = 10240
NC = 2
NS = 16
NW = NC * NS
CH = 128
NCHUNKS = N_EDGES // CH
DH = D // 2
_RPT = N_PAD // NS

_DEG_Q, _DEG_R = divmod(NCHUNKS, NW)
_AGG_Q, _AGG_R = divmod(NCHUNKS, NS)

_mesh = plsc.VectorSubcoreMesh(core_axis_name="c", subcore_axis_name="s")
_sc_params = pltpu.CompilerParams(use_tc_tiling_on_sc=False)


@functools.partial(
    pl.kernel,
    out_type=jax.ShapeDtypeStruct((NC, N_PAD), jnp.float32),
    mesh=_mesh,
    scratch_types=[
        pltpu.VMEM((2, CH), jnp.int32),
        pltpu.VMEM((CH,), jnp.float32),
        pltpu.VMEM((_RPT,), jnp.float32),
        pltpu.VMEM_SHARED((N_PAD,), jnp.float32),
        pltpu.SemaphoreType.DMA,
        pltpu.SemaphoreType.DMA,
    ],
    compiler_params=_sc_params,
)
def _sc_degs(ei_hbm, out_hbm, dst_b, ones_v, zero_v, degs_sh, sem0, sem1):
    c = lax.axis_index("c")
    s = lax.axis_index("s")
    wid = c * NS + s
    r0 = s * _RPT
    nch = _DEG_Q + jnp.where(wid < _DEG_R, 1, 0)
    base = _DEG_Q * wid + jnp.minimum(wid, _DEG_R)

    def fill(i, carry):
        zero_v[pl.ds(i * 16, 16)] = jnp.zeros((16,), jnp.float32)
        return carry

    lax.fori_loop(0, _RPT // 16, fill, 0)
    for i in range(CH // 16):
        ones_v[pl.ds(i * 16, 16)] = jnp.ones((16,), jnp.float32)
    pltpu.sync_copy(zero_v, degs_sh.at[pl.ds(r0, _RPT)])

    def load_idx(chunk, slot, sem):
        pltpu.async_copy(ei_hbm.at[1, pl.ds((base + chunk) * CH, CH)],
                         dst_b.at[slot], sem)

    def wait_idx(sem):
        pltpu.make_async_copy(ei_hbm.at[1, pl.ds(0, CH)], dst_b.at[0], sem).wait()

    load_idx(0, 0, sem0)
    load_idx(1, 1, sem1)
    plsc.subcore_barrier()

    def body(j, carry):
        i0 = 2 * j
        i1 = i0 + 1
        wait_idx(sem0)
        pltpu.sync_copy(ones_v, degs_sh.at[dst_b.at[0]], add=True)

        @pl.when(i0 + 2 < nch)
        def _():
            load_idx(i0 + 2, 0, sem0)

        wait_idx(sem1)
        pltpu.sync_copy(ones_v, degs_sh.at[dst_b.at[1]], add=True)

        @pl.when(i1 + 2 < nch)
        def _():
            load_idx(i1 + 2, 1, sem1)

        return carry

    lax.fori_loop(0, _DEG_Q // 2, body, 0)

    @pl.when(nch > _DEG_Q)
    def _():
        wait_idx(sem0)
        pltpu.sync_copy(ones_v, degs_sh.at[dst_b.at[0]], add=True)

    plsc.subcore_barrier()
    pltpu.sync_copy(degs_sh.at[pl.ds(r0, _RPT)],
                    out_hbm.at[c, pl.ds(r0, _RPT)])


@functools.partial(
    pl.kernel,
    out_type=jax.ShapeDtypeStruct((N_PAD, D), jnp.float32),
    mesh=_mesh,
    scratch_types=[
        pltpu.VMEM((4, CH), jnp.int32),
        pltpu.VMEM((4, CH), jnp.int32),
        pltpu.VMEM((CH, DH), jnp.float32),
        pltpu.VMEM((CH, DH), jnp.float32),
        pltpu.VMEM((CH, DH), jnp.float32),
        pltpu.VMEM((CH, DH), jnp.float32),
        pltpu.VMEM_SHARED((N_PAD, DH), jnp.float32),
        pltpu.VMEM_SHARED((N_PAD, DH), jnp.float32),
        pltpu.SemaphoreType.DMA,
        pltpu.SemaphoreType.DMA,
        pltpu.SemaphoreType.DMA,
        pltpu.SemaphoreType.DMA,
        pltpu.SemaphoreType.DMA,
        pltpu.SemaphoreType.DMA,
        pltpu.SemaphoreType.DMA,
        pltpu.SemaphoreType.DMA,
        pltpu.SemaphoreType.DMA,
        pltpu.SemaphoreType.DMA,
    ],
    compiler_params=_sc_params,
)
def _sc_agg(h_hbm, ei_hbm, out_hbm,
            src_b, dst_b, b0, b1, b2, b3, h_sh, acc_sh,
            g0, g1, g2, g3, s0, s1, s2, s3, sem_i01, sem_i23):
    c = lax.axis_index("c")
    s = lax.axis_index("s")
    r0 = s * _RPT
    bufs = (b0, b1, b2, b3)
    gsems = (g0, g1, g2, g3)
    ssems = (s0, s1, s2, s3)
    nch = _AGG_Q + jnp.where(s < _AGG_R, 1, 0)
    base = _AGG_Q * s + jnp.minimum(s, _AGG_R)

    def fill(i, carry):
        for k in range(DH // 16):
            b0[i, pl.ds(k * 16, 16)] = jnp.zeros((16,), jnp.float32)
        return carry

    lax.fori_loop(0, CH, fill, 0)
    for k in range(_RPT // CH):
        pltpu.sync_copy(b0, acc_sh.at[pl.ds(r0 + k * CH, CH)])
    pltpu.sync_copy(h_hbm.at[pl.ds(r0, _RPT), pl.ds(c * DH, DH)],
                    h_sh.at[pl.ds(r0, _RPT)])

    def load_idx(chunk, slot, sem):
        off = (base + chunk) * CH
        pltpu.async_copy(ei_hbm.at[0, pl.ds(off, CH)], src_b.at[slot], sem)
        pltpu.async_copy(ei_hbm.at[1, pl.ds(off, CH)], dst_b.at[slot], sem)

    def wait_idx(sem, n):
        for _ in range(n):
            pltpu.make_async_copy(ei_hbm.at[0, pl.ds(0, CH)],
                                  src_b.at[0], sem).wait()

    def gather(slot, buf, sem):
        pltpu.async_copy(h_sh.at[src_b.at[slot]], buf, sem)

    def wait_gather(buf, sem):
        pltpu.make_async_copy(h_sh.at[src_b.at[0]], buf, sem).wait()

    def scatter(slot, buf, sem):
        pltpu.async_copy(buf, acc_sh.at[dst_b.at[slot]], sem, add=True)

    def wait_scatter(buf, sem):
        pltpu.make_async_copy(buf, acc_sh.at[dst_b.at[0]], sem).wait()

    load_idx(0, 0, sem_i01)
    load_idx(1, 1, sem_i01)
    load_idx(2, 2, sem_i23)
    load_idx(3, 3, sem_i23)
    plsc.subcore_barrier()

    NJ = _AGG_Q // 4

    def body(j, carry):
        c0 = 4 * j
        wait_idx(sem_i01, 4)
        gather(0, b0, g0)
        gather(1, b1, g1)
        wait_idx(sem_i23, 4)
        gather(2, b2, g2)
        gather(3, b3, g3)
        wait_gather(b0, g0)
        scatter(0, b0, s0)
        wait_gather(b1, g1)
        scatter(1, b1, s1)
        wait_gather(b2, g2)
        scatter(2, b2, s2)
        wait_gather(b3, g3)
        scatter(3, b3, s3)
        wait_scatter(b0, s0)
        wait_scatter(b1, s1)

        @pl.when(c0 + 4 < nch)
        def _():
            load_idx(c0 + 4, 0, sem_i01)

        @pl.when(c0 + 5 < nch)
        def _():
            load_idx(c0 + 5, 1, sem_i01)

        wait_scatter(b2, s2)
        wait_scatter(b3, s3)

        @pl.when(c0 + 6 < nch)
        def _():
            load_idx(c0 + 6, 2, sem_i23)

        @pl.when(c0 + 7 < nch)
        def _():
            load_idx(c0 + 7, 3, sem_i23)

        return carry

    lax.fori_loop(0, NJ, body, 0)

    @pl.when(nch > _AGG_Q)
    def _():
        wait_idx(sem_i01, 2)
        gather(0, b0, g0)
        wait_gather(b0, g0)
        scatter(0, b0, s0)
        wait_scatter(b0, s0)

    plsc.subcore_barrier()
    pltpu.sync_copy(acc_sh.at[pl.ds(r0, _RPT)],
                    out_hbm.at[pl.ds(r0, _RPT), pl.ds(c * DH, DH)])


def _tc_prescale_body(degs_ref, feat_ref, h_ref):
    d = degs_ref[0] + degs_ref[1]
    norm = jnp.where(d > 0.0, lax.rsqrt(d), 0.0)
    h_ref[:N_NODES] = feat_ref[...] * norm[:N_NODES]
    h_ref[N_NODES:] = jnp.zeros((N_PAD - N_NODES, D), jnp.float32)


def _tc_finish_body(agg_ref, degs_ref, w_ref, b_ref, out_ref):
    a = agg_ref[:N_NODES]
    d = degs_ref[0, :N_NODES] + degs_ref[1, :N_NODES]
    norm = jnp.where(d > 0.0, lax.rsqrt(d), 0.0)
    h2 = a * norm
    out = lax.dot_general(h2, w_ref[...],
                          dimension_numbers=(((1,), (1,)), ((), ())),
                          preferred_element_type=jnp.float32)
    out_ref[...] = out + b_ref[...]


def kernel(features, edge_index, W, b):
    features = features.astype(jnp.float32)
    ei = edge_index.astype(jnp.int32)

    degs_p = _sc_degs(ei)
    degs_p3 = degs_p[:, :, None]

    h = pl.pallas_call(
        _tc_prescale_body,
        out_shape=jax.ShapeDtypeStruct((N_PAD, D), jnp.float32),
    )(degs_p3, features)

    agg = _sc_agg(h, ei)

    out = pl.pallas_call(
        _tc_finish_body,
        out_shape=jax.ShapeDtypeStruct((N_NODES, D), jnp.float32),
    )(agg, degs_p3, W.astype(jnp.float32), b.reshape(1, D).astype(jnp.float32))
    return out
